# baseline (speedup 1.0000x reference)
import functools

import jax
import jax.numpy as jnp
from jax import lax
from jax.experimental import pallas as pl
from jax.experimental.pallas import tpu as pltpu
from jax.experimental.pallas import tpu_sc as plsc

N_N = 50000
N_H = 10000
NNZ = 800000
C = 64

CH = 512
EPT_A = 25600
ITER_A = EPT_A // 1024
RPAD = 32 * EPT_A
RROWS = RPAD // 128
EPT_B = 2 * EPT_A
ITER_B = 2 * ITER_A

CNT_CH = 1024
CNT_PAD = 819200
CNT_EPT = CNT_PAD // 32
CNT_CPT = CNT_EPT // CNT_CH
SS_EPT = CNT_PAD // 16
SS_CPT = SS_EPT // CNT_CH

HN = 10112
NN = 50016
DBH = N_N // 2
DBA = 25088


def _mesh():
    return plsc.VectorSubcoreMesh(core_axis_name="c", subcore_axis_name="s")


def _zero16():
    return jnp.zeros((16,), jnp.float32)


@functools.cache
def _sc_counts():
    @functools.partial(
        pl.kernel,
        out_type=(
            jax.ShapeDtypeStruct((32 * HN,), jnp.float32),
            jax.ShapeDtypeStruct((32 * NN,), jnp.float32),
        ),
        mesh=_mesh(),
        compiler_params=pltpu.CompilerParams(needs_layout_passes=False, use_tc_tiling_on_sc=False),
        scratch_types=[
            pltpu.VMEM((HN,), jnp.float32),
            pltpu.VMEM((NN,), jnp.float32),
            pltpu.VMEM((CNT_CH,), jnp.int32),
            pltpu.VMEM((CNT_CH,), jnp.int32),
        ],
    )
    def k(hc_hbm, nc_hbm, de_p, dv_p, acc_de, acc_dv, hbuf, nbuf):
        c = lax.axis_index("c")
        s = lax.axis_index("s")
        w = c * 16 + s
        z = _zero16()
        ones = jnp.ones((16,), jnp.float32)

        @pl.loop(0, HN // 16)
        def _(i):
            acc_de[pl.ds(i * 16, 16)] = z

        @pl.loop(0, NN // 16)
        def _(i):
            acc_dv[pl.ds(i * 16, 16)] = z

        @pl.loop(0, CNT_CPT)
        def _(kk):
            b = w * CNT_EPT + kk * CNT_CH
            pltpu.sync_copy(hc_hbm.at[pl.ds(b, CNT_CH)], hbuf)
            pltpu.sync_copy(nc_hbm.at[pl.ds(b, CNT_CH)], nbuf)

            @pl.loop(0, CNT_CH // 16)
            def _(i):
                hv = hbuf[pl.ds(i * 16, 16)]
                plsc.addupdate_scatter(acc_de, [hv], ones)
                nv = nbuf[pl.ds(i * 16, 16)]
                plsc.addupdate_scatter(acc_dv, [nv], ones)

        pltpu.sync_copy(acc_de, de_p.at[pl.ds(w * HN, HN)])
        pltpu.sync_copy(acc_dv, dv_p.at[pl.ds(w * NN, NN)])

    return k


@functools.cache
def _sc_ssums():
    @functools.partial(
        pl.kernel,
        out_type=(
            jax.ShapeDtypeStruct((16 * NN,), jnp.float32),
            jax.ShapeDtypeStruct((16 * HN,), jnp.float32),
        ),
        mesh=_mesh(),
        compiler_params=pltpu.CompilerParams(needs_layout_passes=False, use_tc_tiling_on_sc=False),
        scratch_types=[
            pltpu.VMEM((HN,), jnp.float32),
            pltpu.VMEM((NN,), jnp.float32),
            pltpu.VMEM((HN,), jnp.float32),
            pltpu.VMEM((NN,), jnp.float32),
            pltpu.VMEM((CNT_CH,), jnp.int32),
            pltpu.VMEM((CNT_CH,), jnp.int32),
        ],
    )
    def k(hc_hbm, nc_hbm, dea_hbm, dvb_hbm, s0_p, s1_p,
          tbl_s, tbl_b, acc_s, acc_b, hbuf, nbuf):
        c = lax.axis_index("c")
        s = lax.axis_index("s")
        z = _zero16()

        @pl.when(c == 0)
        def _():
            pltpu.sync_copy(dea_hbm, tbl_s)

            @pl.loop(0, NN // 16)
            def _(i):
                acc_b[pl.ds(i * 16, 16)] = z

            @pl.loop(0, SS_CPT)
            def _(kk):
                b = s * SS_EPT + kk * CNT_CH
                pltpu.sync_copy(hc_hbm.at[pl.ds(b, CNT_CH)], hbuf)
                pltpu.sync_copy(nc_hbm.at[pl.ds(b, CNT_CH)], nbuf)

                @pl.loop(0, CNT_CH // 16)
                def _(i):
                    hv = hbuf[pl.ds(i * 16, 16)]
                    nv = nbuf[pl.ds(i * 16, 16)]
                    val = plsc.load_gather(tbl_s, [hv])
                    plsc.addupdate_scatter(acc_b, [nv], val)

            pltpu.sync_copy(acc_b, s0_p.at[pl.ds(s * NN, NN)])

        @pl.when(c == 1)
        def _():
            pltpu.sync_copy(dvb_hbm, tbl_b)

            @pl.loop(0, HN // 16)
            def _(i):
                acc_s[pl.ds(i * 16, 16)] = z

            @pl.loop(0, SS_CPT)
            def _(kk):
                b = s * SS_EPT + kk * CNT_CH
                pltpu.sync_copy(hc_hbm.at[pl.ds(b, CNT_CH)], hbuf)
                pltpu.sync_copy(nc_hbm.at[pl.ds(b, CNT_CH)], nbuf)

                @pl.loop(0, CNT_CH // 16)
                def _(i):
                    hv = hbuf[pl.ds(i * 16, 16)]
                    nv = nbuf[pl.ds(i * 16, 16)]
                    val = plsc.load_gather(tbl_b, [nv])
                    plsc.addupdate_scatter(acc_s, [hv], val)

            pltpu.sync_copy(acc_s, s1_p.at[pl.ds(s * HN, HN)])

    return k


def _zero_rows(rows_v, n):
    z = _zero16()

    @pl.loop(0, n)
    def _(i):
        for q in range(4):
            rows_v[i, pl.ds(q * 16, 16)] = z


@functools.cache
def _sc_seg_hedge():
    @functools.partial(
        pl.kernel,
        out_type=jax.ShapeDtypeStruct((2 * HN, C), jnp.float32),
        mesh=_mesh(),
        compiler_params=pltpu.CompilerParams(needs_layout_passes=False, use_tc_tiling_on_sc=False),
        scratch_types=[
            pltpu.VMEM((5120,), jnp.int32),
            pltpu.VMEM((40, 128), jnp.int32),
            pltpu.VMEM((1024, C), jnp.float32),
            pltpu.VMEM_SHARED((HN, C), jnp.float32),
            pltpu.SemaphoreType.DMA,
            pltpu.SemaphoreType.DMA,
        ],
    )
    def k(in_hbm, src_hbm, dst_hbm, out_hbm,
          src_v, idx_v, rows_v, acc, semg, sems):
        c = lax.axis_index("c")
        s = lax.axis_index("s")
        w = c * 16 + s

        _zero_rows(rows_v, 256)
        st = s * (HN // 16)
        for t in range(2):
            pltpu.sync_copy(rows_v.at[pl.ds(0, 256)],
                            acc.at[pl.ds(st + t * 256, 256)])
        pltpu.sync_copy(rows_v.at[pl.ds(0, 120)], acc.at[pl.ds(st + 512, 120)])
        plsc.subcore_barrier()

        @pl.loop(0, ITER_A // 5)
        def _(u):
            eb = w * EPT_A + u * 5120
            rb = w * (EPT_A // 128) + u * 40
            d1 = pltpu.async_copy(src_hbm.at[pl.ds(eb, 5120)], src_v, semg)
            d2 = pltpu.async_copy(dst_hbm.at[pl.ds(rb, 40)], idx_v, semg)
            d1.wait()
            d2.wait()
            _pipe_super(in_hbm, acc, src_v, idx_v, rows_v, semg, sems, 512)

        plsc.subcore_barrier()
        pltpu.sync_copy(acc.at[pl.ds(st, CH)],
                        out_hbm.at[pl.ds(c * HN + st, CH)])
        pltpu.sync_copy(acc.at[pl.ds(st + CH, HN // 16 - CH)],
                        out_hbm.at[pl.ds(c * HN + st + CH, HN // 16 - CH)])

    return k


def _pipe_super(in_hbm, acc, src_v, idx_v, rows_v, semg, sems, psz):
    npart = 5120 // psz
    nsc = psz // 128

    def issue_gather(p, off):
        pltpu.async_copy(in_hbm.at[src_v.at[pl.ds(p * psz, psz)]],
                         rows_v.at[pl.ds(off, psz)], semg)

    def wait_gather(p, off):
        pltpu.make_async_copy(in_hbm.at[src_v.at[pl.ds(p * psz, psz)]],
                              rows_v.at[pl.ds(off, psz)], semg).wait()

    def issue_scatters(p, off):
        for j in range(nsc):
            pltpu.async_copy(rows_v.at[pl.ds(off + j * 128, 128)],
                             acc.at[idx_v.at[p * nsc + j]], sems, add=True)

    def wait_scatters(p, off):
        for j in range(nsc):
            pltpu.make_async_copy(rows_v.at[pl.ds(off + j * 128, 128)],
                                  acc.at[idx_v.at[p * nsc + j]],
                                  sems).wait()

    issue_gather(0, 0)

    @pl.loop(0, npart)
    def _(p):
        b = (p % 2) * psz
        nb = psz - b

        @pl.when(p >= 1)
        def _():
            wait_scatters(p - 1, nb)

        @pl.when(p < npart - 1)
        def _():
            issue_gather(p + 1, nb)

        wait_gather(p, b)
        issue_scatters(p, b)

    wait_scatters(npart - 1, ((npart - 1) % 2) * psz)


@functools.cache
def _sc_seg_node():
    @functools.partial(
        pl.kernel,
        out_type=jax.ShapeDtypeStruct((N_N, C), jnp.float32),
        mesh=_mesh(),
        compiler_params=pltpu.CompilerParams(needs_layout_passes=False, use_tc_tiling_on_sc=False),
        scratch_types=[
            pltpu.VMEM((5120,), jnp.int32),
            pltpu.VMEM((40, 128), jnp.int32),
            pltpu.VMEM((256, C), jnp.float32),
            pltpu.VMEM_SHARED((DBA, C), jnp.float32),
            pltpu.SemaphoreType.DMA,
            pltpu.SemaphoreType.DMA,
        ],
    )
    def k(in_hbm, src_hbm, dlo_hbm, dhi_hbm, out_hbm,
          src_v, idx_v, rows_v, acc, semg, sems):
        c = lax.axis_index("c")
        s = lax.axis_index("s")
        lo = c * DBH

        _zero_rows(rows_v, 256)
        st = s * (DBA // 16)
        for t in range(6):
            pltpu.sync_copy(rows_v, acc.at[pl.ds(st + t * 256, 256)])
        pltpu.sync_copy(rows_v.at[pl.ds(0, 32)], acc.at[pl.ds(st + 1536, 32)])
        plsc.subcore_barrier()

        def main(dst_hbm):
            @pl.loop(0, ITER_B // 5)
            def _(u):
                eb = s * EPT_B + u * 5120
                rb = s * (EPT_B // 128) + u * 40
                d1 = pltpu.async_copy(src_hbm.at[pl.ds(eb, 5120)], src_v,
                                      semg)
                d2 = pltpu.async_copy(dst_hbm.at[pl.ds(rb, 40)], idx_v, semg)
                d1.wait()
                d2.wait()
                _pipe_super(in_hbm, acc, src_v, idx_v, rows_v, semg, sems,
                            128)

        @pl.when(c == 0)
        def _():
            main(dlo_hbm)

        @pl.when(c == 1)
        def _():
            main(dhi_hbm)

        plsc.subcore_barrier()
        st15 = s * 1560
        for t in range(3):
            pltpu.sync_copy(acc.at[pl.ds(st15 + t * CH, CH)],
                            out_hbm.at[pl.ds(lo + st15 + t * CH, CH)])
        pltpu.sync_copy(acc.at[pl.ds(st15 + 3 * CH, 24)],
                        out_hbm.at[pl.ds(lo + st15 + 3 * CH, 24)])

        @pl.when(s == 0)
        def _():
            pltpu.sync_copy(acc.at[pl.ds(24960, 40)],
                            out_hbm.at[pl.ds(lo + 24960, 40)])

    return k


@functools.cache
def _tc_scales():
    def body(de_ref, dv_ref, dea_ref, dvb_ref):
        de = jnp.sum(de_ref[...], axis=0, keepdims=True)
        r = lax.rsqrt(de)
        dea_ref[...] = jnp.where(de > 0, r * r * r, 0.0)
        dv = jnp.sum(dv_ref[...], axis=0, keepdims=True)
        r2 = lax.rsqrt(dv)
        dvb_ref[...] = jnp.where(dv > 0, r2, 0.0)

    return pl.pallas_call(
        body,
        out_shape=(
            jax.ShapeDtypeStruct((1, HN), jnp.float32),
            jax.ShapeDtypeStruct((1, NN), jnp.float32),
        ),
    )


@functools.cache
def _tc_pre():
    blk = 2000

    def body(x_ref, sc_ref, w_ref, o_ref):
        o_ref[...] = jnp.dot(x_ref[...] * sc_ref[...], w_ref[...],
                             preferred_element_type=jnp.float32)

    return pl.pallas_call(
        body,
        grid=(N_N // blk,),
        in_specs=[
            pl.BlockSpec((blk, C), lambda i: (i, 0)),
            pl.BlockSpec((blk, 1), lambda i: (i, 0)),
            pl.BlockSpec((C, C), lambda i: (0, 0)),
        ],
        out_specs=pl.BlockSpec((blk, C), lambda i: (i, 0)),
        out_shape=jax.ShapeDtypeStruct((N_N, C), jnp.float32),
    )


@functools.cache
def _tc_hedge():
    blk = 2000

    def body(pa_ref, sp_ref, dea_ref, b_ref, w_ref, o_ref):
        seg = pa_ref[0] + pa_ref[1]
        s1 = jnp.sum(sp_ref[...], axis=1, keepdims=True)
        inv = jnp.where(s1 > 0, 1.0 / s1, 0.0)
        x1 = jnp.maximum(seg * inv + b_ref[...], 0.0)
        o_ref[...] = jnp.dot(x1 * dea_ref[...], w_ref[...],
                             preferred_element_type=jnp.float32)

    return pl.pallas_call(
        body,
        grid=(N_H // blk,),
        in_specs=[
            pl.BlockSpec((2, blk, C), lambda i: (0, i, 0)),
            pl.BlockSpec((blk, 16), lambda i: (i, 0)),
            pl.BlockSpec((blk, 1), lambda i: (i, 0)),
            pl.BlockSpec((1, C), lambda i: (0, 0)),
            pl.BlockSpec((C, C), lambda i: (0, 0)),
        ],
        out_specs=pl.BlockSpec((blk, C), lambda i: (i, 0)),
        out_shape=jax.ShapeDtypeStruct((N_H, C), jnp.float32),
    )


@functools.cache
def _tc_node():
    blk = 2000

    def body(seg_ref, sp_ref, b_ref, sc_ref, w_ref, o_ref):
        s0 = jnp.sum(sp_ref[...], axis=1, keepdims=True)
        inv = jnp.where(s0 > 0, 1.0 / s0, 0.0)
        x = jnp.maximum(seg_ref[...] * inv + b_ref[...], 0.0)
        o_ref[...] = jnp.dot(x * sc_ref[...], w_ref[...],
                             preferred_element_type=jnp.float32)

    return pl.pallas_call(
        body,
        grid=(N_N // blk,),
        in_specs=[
            pl.BlockSpec((blk, C), lambda i: (i, 0)),
            pl.BlockSpec((blk, 16), lambda i: (i, 0)),
            pl.BlockSpec((1, C), lambda i: (0, 0)),
            pl.BlockSpec((blk, 1), lambda i: (i, 0)),
            pl.BlockSpec((C, C), lambda i: (0, 0)),
        ],
        out_specs=pl.BlockSpec((blk, C), lambda i: (i, 0)),
        out_shape=jax.ShapeDtypeStruct((N_N, C), jnp.float32),
    )


@functools.cache
def _tc_final():
    blk = 2000
    ngrid = N_N // blk

    def body(seg_ref, sp_ref, b_ref, wl_ref, bl_ref, o_ref, pool_ref):
        s0 = jnp.sum(sp_ref[...], axis=1, keepdims=True)
        inv = jnp.where(s0 > 0, 1.0 / s0, 0.0)
        x = jnp.maximum(seg_ref[...] * inv + b_ref[...], 0.0)
        bm = jnp.max(x, axis=0, keepdims=True)
        i = pl.program_id(0)

        @pl.when(i == 0)
        def _():
            pool_ref[...] = bm

        @pl.when(i > 0)
        def _():
            pool_ref[...] = jnp.maximum(pool_ref[...], bm)

        @pl.when(i == ngrid - 1)
        def _():
            o_ref[...] = jnp.dot(pool_ref[...], wl_ref[...],
                                 preferred_element_type=jnp.float32) + bl_ref[...]

    return pl.pallas_call(
        body,
        grid=(ngrid,),
        in_specs=[
            pl.BlockSpec((blk, C), lambda i: (i, 0)),
            pl.BlockSpec((blk, 16), lambda i: (i, 0)),
            pl.BlockSpec((1, C), lambda i: (0, 0)),
            pl.BlockSpec((C, 1), lambda i: (0, 0)),
            pl.BlockSpec((1, 1), lambda i: (0, 0)),
        ],
        out_specs=pl.BlockSpec((1, 1), lambda i: (0, 0)),
        out_shape=jax.ShapeDtypeStruct((1, 1), jnp.float32),
        scratch_shapes=[pltpu.VMEM((1, C), jnp.float32)],
    )


def _pad_to(a, n, val):
    return jnp.concatenate(
        [a, jnp.full((n - a.shape[0],), val, a.dtype)])


def kernel(x_0, node_idx, hedge_idx,
           W01_1, b01_1, W10_1, b10_1,
           W01_2, b01_2, W10_2, b10_2,
           W_lin, b_lin):
    ni = node_idx.astype(jnp.int32)
    hi = hedge_idx.astype(jnp.int32)

    src_a = _pad_to(ni, RPAD, 0)
    dst_a = _pad_to(hi, RPAD, N_H).reshape(RROWS, 128)
    src_b = _pad_to(hi, RPAD, 0)
    nip = _pad_to(ni, RPAD, N_N)
    dst_blo = jnp.where(nip < DBH, nip, DBH).reshape(RROWS, 128)
    nih = nip - DBH
    dst_bhi = jnp.where((nih >= 0) & (nih < DBH), nih, DBH).reshape(RROWS, 128)
    hc = _pad_to(hi, CNT_PAD, N_H)
    nc = _pad_to(ni, CNT_PAD, N_N)

    de_p, dv_p = _sc_counts()(hc, nc)
    dea_t, dvb_t = _tc_scales()(de_p.reshape(32, HN), dv_p.reshape(32, NN))
    s0_p, s1_p = _sc_ssums()(hc, nc, dea_t[0], dvb_t[0])
    s0_p = s0_p.reshape(16, NN)
    s1_p = s1_p.reshape(16, HN)

    s0t = s0_p[:, :N_N].T
    s1t = s1_p[:, :N_H].T
    dea_col = dea_t[0, :N_H].reshape(N_H, 1)
    dvb_col = dvb_t[0, :N_N].reshape(N_N, 1)
    b01_1r = b01_1.reshape(1, C)
    b10_1r = b10_1.reshape(1, C)
    b01_2r = b01_2.reshape(1, C)
    b10_2r = b10_2.reshape(1, C)

    m = _tc_pre()(x_0, dvb_col, W01_1)
    pa = _sc_seg_hedge()(m, src_a, dst_a).reshape(2, HN, C)[:, :N_H]
    m1 = _tc_hedge()(pa, s1t, dea_col, b01_1r, W10_1)
    segb = _sc_seg_node()(m1, src_b, dst_blo, dst_bhi)
    m2 = _tc_node()(segb, s0t, b10_1r, dvb_col, W01_2)
    pa2 = _sc_seg_hedge()(m2, src_a, dst_a).reshape(2, HN, C)[:, :N_H]
    m3 = _tc_hedge()(pa2, s1t, dea_col, b01_2r, W10_2)
    segb2 = _sc_seg_node()(m3, src_b, dst_blo, dst_bhi)
    out = _tc_final()(segb2, s0t, b10_2r, W_lin, b_lin.reshape(1, 1))
    return out.reshape(1)


# speedup vs baseline: 1.4617x; 1.4617x over previous
import functools

import jax
import jax.numpy as jnp
from jax import lax
from jax.experimental import pallas as pl
from jax.experimental.pallas import tpu as pltpu
from jax.experimental.pallas import tpu_sc as plsc

N_N = 50000
N_H = 10000
NNZ = 800000
C = 64

CH = 512
EPT_A = 25600
ITER_A = EPT_A // 1024
RPAD = 32 * EPT_A
RROWS = RPAD // 128
EPT_B = 2 * EPT_A
ITER_B = 2 * ITER_A

CNT_CH = 1024
CNT_PAD = 819200
CNT_EPT = CNT_PAD // 32
CNT_CPT = CNT_EPT // CNT_CH
SS_EPT = CNT_PAD // 16
SS_CPT = SS_EPT // CNT_CH

HN = 10112
NN = 50016
DBH = N_N // 2
DBA = 25088


def _mesh():
    return plsc.VectorSubcoreMesh(core_axis_name="c", subcore_axis_name="s")


def _zero16():
    return jnp.zeros((16,), jnp.float32)


@functools.cache
def _sc_counts():
    @functools.partial(
        pl.kernel,
        out_type=(
            jax.ShapeDtypeStruct((32 * HN,), jnp.float32),
            jax.ShapeDtypeStruct((32 * NN,), jnp.float32),
        ),
        mesh=_mesh(),
        compiler_params=pltpu.CompilerParams(needs_layout_passes=False, use_tc_tiling_on_sc=False),
        scratch_types=[
            pltpu.VMEM((HN,), jnp.float32),
            pltpu.VMEM((NN,), jnp.float32),
            pltpu.VMEM((CNT_CH,), jnp.int32),
            pltpu.VMEM((CNT_CH,), jnp.int32),
        ],
    )
    def k(hc_hbm, nc_hbm, de_p, dv_p, acc_de, acc_dv, hbuf, nbuf):
        c = lax.axis_index("c")
        s = lax.axis_index("s")
        w = c * 16 + s
        z = _zero16()
        ones = jnp.ones((16,), jnp.float32)

        @pl.loop(0, HN // 16)
        def _(i):
            acc_de[pl.ds(i * 16, 16)] = z

        @pl.loop(0, NN // 16)
        def _(i):
            acc_dv[pl.ds(i * 16, 16)] = z

        @pl.loop(0, CNT_CPT)
        def _(kk):
            b = w * CNT_EPT + kk * CNT_CH
            pltpu.sync_copy(hc_hbm.at[pl.ds(b, CNT_CH)], hbuf)
            pltpu.sync_copy(nc_hbm.at[pl.ds(b, CNT_CH)], nbuf)

            @pl.loop(0, CNT_CH // 16)
            def _(i):
                hv = hbuf[pl.ds(i * 16, 16)]
                plsc.addupdate_scatter(acc_de, [hv], ones)
                nv = nbuf[pl.ds(i * 16, 16)]
                plsc.addupdate_scatter(acc_dv, [nv], ones)

        pltpu.sync_copy(acc_de, de_p.at[pl.ds(w * HN, HN)])
        pltpu.sync_copy(acc_dv, dv_p.at[pl.ds(w * NN, NN)])

    return k


@functools.cache
def _sc_ssums():
    @functools.partial(
        pl.kernel,
        out_type=(
            jax.ShapeDtypeStruct((16 * NN,), jnp.float32),
            jax.ShapeDtypeStruct((16 * HN,), jnp.float32),
        ),
        mesh=_mesh(),
        compiler_params=pltpu.CompilerParams(needs_layout_passes=False, use_tc_tiling_on_sc=False),
        scratch_types=[
            pltpu.VMEM((HN,), jnp.float32),
            pltpu.VMEM((NN,), jnp.float32),
            pltpu.VMEM((HN,), jnp.float32),
            pltpu.VMEM((NN,), jnp.float32),
            pltpu.VMEM((CNT_CH,), jnp.int32),
            pltpu.VMEM((CNT_CH,), jnp.int32),
        ],
    )
    def k(hc_hbm, nc_hbm, dea_hbm, dvb_hbm, s0_p, s1_p,
          tbl_s, tbl_b, acc_s, acc_b, hbuf, nbuf):
        c = lax.axis_index("c")
        s = lax.axis_index("s")
        z = _zero16()

        @pl.when(c == 0)
        def _():
            pltpu.sync_copy(dea_hbm, tbl_s)

            @pl.loop(0, NN // 16)
            def _(i):
                acc_b[pl.ds(i * 16, 16)] = z

            @pl.loop(0, SS_CPT)
            def _(kk):
                b = s * SS_EPT + kk * CNT_CH
                pltpu.sync_copy(hc_hbm.at[pl.ds(b, CNT_CH)], hbuf)
                pltpu.sync_copy(nc_hbm.at[pl.ds(b, CNT_CH)], nbuf)

                @pl.loop(0, CNT_CH // 16)
                def _(i):
                    hv = hbuf[pl.ds(i * 16, 16)]
                    nv = nbuf[pl.ds(i * 16, 16)]
                    val = plsc.load_gather(tbl_s, [hv])
                    plsc.addupdate_scatter(acc_b, [nv], val)

            pltpu.sync_copy(acc_b, s0_p.at[pl.ds(s * NN, NN)])

        @pl.when(c == 1)
        def _():
            pltpu.sync_copy(dvb_hbm, tbl_b)

            @pl.loop(0, HN // 16)
            def _(i):
                acc_s[pl.ds(i * 16, 16)] = z

            @pl.loop(0, SS_CPT)
            def _(kk):
                b = s * SS_EPT + kk * CNT_CH
                pltpu.sync_copy(hc_hbm.at[pl.ds(b, CNT_CH)], hbuf)
                pltpu.sync_copy(nc_hbm.at[pl.ds(b, CNT_CH)], nbuf)

                @pl.loop(0, CNT_CH // 16)
                def _(i):
                    hv = hbuf[pl.ds(i * 16, 16)]
                    nv = nbuf[pl.ds(i * 16, 16)]
                    val = plsc.load_gather(tbl_b, [nv])
                    plsc.addupdate_scatter(acc_s, [hv], val)

            pltpu.sync_copy(acc_s, s1_p.at[pl.ds(s * HN, HN)])

    return k


def _zero_rows(rows_v, n):
    z = _zero16()

    @pl.loop(0, n)
    def _(i):
        for q in range(4):
            rows_v[i, pl.ds(q * 16, 16)] = z


@functools.cache
def _sc_seg_hedge():
    @functools.partial(
        pl.kernel,
        out_type=jax.ShapeDtypeStruct((2 * HN, C), jnp.float32),
        mesh=_mesh(),
        compiler_params=pltpu.CompilerParams(needs_layout_passes=False, use_tc_tiling_on_sc=False),
        scratch_types=[
            pltpu.VMEM((5120,), jnp.int32),
            pltpu.VMEM((40, 128), jnp.int32),
            pltpu.VMEM((1024, C), jnp.float32),
            pltpu.VMEM_SHARED((HN, C), jnp.float32),
            pltpu.SemaphoreType.DMA,
            pltpu.SemaphoreType.DMA,
        ],
    )
    def k(in_hbm, src_hbm, dst_hbm, out_hbm,
          src_v, idx_v, rows_v, acc, semg, sems):
        c = lax.axis_index("c")
        s = lax.axis_index("s")
        w = c * 16 + s

        _zero_rows(rows_v, 256)
        st = s * (HN // 16)
        for t in range(2):
            pltpu.sync_copy(rows_v.at[pl.ds(0, 256)],
                            acc.at[pl.ds(st + t * 256, 256)])
        pltpu.sync_copy(rows_v.at[pl.ds(0, 120)], acc.at[pl.ds(st + 512, 120)])
        plsc.subcore_barrier()

        @pl.loop(0, ITER_A // 5)
        def _(u):
            eb = w * EPT_A + u * 5120
            rb = w * (EPT_A // 128) + u * 40
            d1 = pltpu.async_copy(src_hbm.at[pl.ds(eb, 5120)], src_v, semg)
            d2 = pltpu.async_copy(dst_hbm.at[pl.ds(rb, 40)], idx_v, semg)
            d1.wait()
            d2.wait()
            _pipe_super(in_hbm, acc, src_v, idx_v, rows_v, semg, sems, 512)

        plsc.subcore_barrier()
        pltpu.sync_copy(acc.at[pl.ds(st, CH)],
                        out_hbm.at[pl.ds(c * HN + st, CH)])
        pltpu.sync_copy(acc.at[pl.ds(st + CH, HN // 16 - CH)],
                        out_hbm.at[pl.ds(c * HN + st + CH, HN // 16 - CH)])

    return k


def _pipe_super(in_hbm, acc, src_v, idx_v, rows_v, semg, sems, psz):
    npart = 5120 // psz
    nsc = psz // 128

    def issue_gather(p, off):
        pltpu.async_copy(in_hbm.at[src_v.at[pl.ds(p * psz, psz)]],
                         rows_v.at[pl.ds(off, psz)], semg)

    def wait_gather(p, off):
        pltpu.make_async_copy(in_hbm.at[src_v.at[pl.ds(p * psz, psz)]],
                              rows_v.at[pl.ds(off, psz)], semg).wait()

    def issue_scatters(p, off):
        for j in range(nsc):
            pltpu.async_copy(rows_v.at[pl.ds(off + j * 128, 128)],
                             acc.at[idx_v.at[p * nsc + j]], sems, add=True)

    def wait_scatters(p, off):
        for j in range(nsc):
            pltpu.make_async_copy(rows_v.at[pl.ds(off + j * 128, 128)],
                                  acc.at[idx_v.at[p * nsc + j]],
                                  sems).wait()

    issue_gather(0, 0)

    @pl.loop(0, npart)
    def _(p):
        b = (p % 2) * psz
        nb = psz - b

        @pl.when(p >= 1)
        def _():
            wait_scatters(p - 1, nb)

        @pl.when(p < npart - 1)
        def _():
            issue_gather(p + 1, nb)

        wait_gather(p, b)
        issue_scatters(p, b)

    wait_scatters(npart - 1, ((npart - 1) % 2) * psz)


@functools.cache
def _sc_seg_node():
    @functools.partial(
        pl.kernel,
        out_type=jax.ShapeDtypeStruct((N_N, C), jnp.float32),
        mesh=_mesh(),
        compiler_params=pltpu.CompilerParams(needs_layout_passes=False, use_tc_tiling_on_sc=False),
        scratch_types=[
            pltpu.VMEM((5248,), jnp.int32),
            pltpu.VMEM((5248,), jnp.int32),
            pltpu.VMEM((256, C), jnp.float32),
            pltpu.VMEM_SHARED((DBA, C), jnp.float32),
            pltpu.SemaphoreType.DMA,
            pltpu.SemaphoreType.DMA,
        ],
    )
    def k(in_hbm, src_hbm, dlo_hbm, dhi_hbm, out_hbm,
          src_v, dst_v, rows_v, acc, semg, sems):
        c = lax.axis_index("c")
        s = lax.axis_index("s")
        lo = c * DBH

        _zero_rows(rows_v, 256)
        st = s * (DBA // 16)
        for t in range(6):
            pltpu.sync_copy(rows_v, acc.at[pl.ds(st + t * 256, 256)])
        pltpu.sync_copy(rows_v.at[pl.ds(0, 32)], acc.at[pl.ds(st + 1536, 32)])
        plsc.subcore_barrier()

        def gat(p, off):
            return pltpu.async_copy(
                in_hbm.at[src_v.at[pl.ds(p * 128, 128)]],
                rows_v.at[pl.ds(off, 128)], semg)

        def wgat(p, off):
            pltpu.make_async_copy(
                in_hbm.at[src_v.at[pl.ds(p * 128, 128)]],
                rows_v.at[pl.ds(off, 128)], semg).wait()

        def sca(p, off):
            return pltpu.async_copy(
                rows_v.at[pl.ds(off, 128)],
                acc.at[dst_v.at[pl.ds(p * 128, 128)]], sems, add=True)

        def wsca(p, off):
            pltpu.make_async_copy(
                rows_v.at[pl.ds(off, 128)],
                acc.at[dst_v.at[pl.ds(p * 128, 128)]], sems).wait()

        def main(dst_hbm):
            @pl.loop(0, ITER_B // 5)
            def _(u):
                eb = s * EPT_B + u * 5120
                d1 = pltpu.async_copy(src_hbm.at[pl.ds(eb, 5120)],
                                      src_v.at[pl.ds(0, 5120)], semg)
                d2 = pltpu.async_copy(dst_hbm.at[pl.ds(eb, 5120)],
                                      dst_v.at[pl.ds(0, 5120)], semg)
                d1.wait()
                d2.wait()

                @pl.loop(0, 40, init_carry=0)
                def n(r, cur):
                    for t in range(8):
                        off = r * 128 + t * 16
                        dv = dst_v[pl.ds(off, 16)]
                        sv = src_v[pl.ds(off, 16)]
                        msk = dv != DBH
                        plsc.store_compressed(dst_v.at[pl.ds(cur, 16)], dv,
                                              mask=msk)
                        plsc.store_compressed(src_v.at[pl.ds(cur, 16)], sv,
                                              mask=msk)
                        cur = cur + plsc.all_reduce_population_count(msk)[0]
                    return cur

                for t in range(8):
                    dst_v[pl.ds(n + t * 16, 16)] = jnp.full((16,), DBH,
                                                            jnp.int32)
                    src_v[pl.ds(n + t * 16, 16)] = jnp.zeros((16,), jnp.int32)
                nparts = (n + 127) // 128

                @pl.when(nparts > 0)
                def _():
                    gat(0, 0)

                    @pl.loop(0, nparts)
                    def _(p):
                        b = (p % 2) * 128
                        nb = 128 - b

                        @pl.when(p >= 1)
                        def _():
                            wsca(p - 1, nb)

                        @pl.when(p < nparts - 1)
                        def _():
                            gat(p + 1, nb)

                        wgat(p, b)
                        sca(p, b)

                    wsca(nparts - 1, ((nparts - 1) % 2) * 128)

        @pl.when(c == 0)
        def _():
            main(dlo_hbm)

        @pl.when(c == 1)
        def _():
            main(dhi_hbm)

        plsc.subcore_barrier()
        st15 = s * 1560
        for t in range(3):
            pltpu.sync_copy(acc.at[pl.ds(st15 + t * CH, CH)],
                            out_hbm.at[pl.ds(lo + st15 + t * CH, CH)])
        pltpu.sync_copy(acc.at[pl.ds(st15 + 3 * CH, 24)],
                        out_hbm.at[pl.ds(lo + st15 + 3 * CH, 24)])

        @pl.when(s == 0)
        def _():
            pltpu.sync_copy(acc.at[pl.ds(24960, 40)],
                            out_hbm.at[pl.ds(lo + 24960, 40)])

    return k


@functools.cache
def _tc_scales():
    def body(de_ref, dv_ref, dea_ref, dvb_ref):
        de = jnp.sum(de_ref[...], axis=0, keepdims=True)
        r = lax.rsqrt(de)
        dea_ref[...] = jnp.where(de > 0, r * r * r, 0.0)
        dv = jnp.sum(dv_ref[...], axis=0, keepdims=True)
        r2 = lax.rsqrt(dv)
        dvb_ref[...] = jnp.where(dv > 0, r2, 0.0)

    return pl.pallas_call(
        body,
        out_shape=(
            jax.ShapeDtypeStruct((1, HN), jnp.float32),
            jax.ShapeDtypeStruct((1, NN), jnp.float32),
        ),
    )


@functools.cache
def _tc_pre():
    blk = 2000

    def body(x_ref, sc_ref, w_ref, o_ref):
        o_ref[...] = jnp.dot(x_ref[...] * sc_ref[...], w_ref[...],
                             preferred_element_type=jnp.float32)

    return pl.pallas_call(
        body,
        grid=(N_N // blk,),
        in_specs=[
            pl.BlockSpec((blk, C), lambda i: (i, 0)),
            pl.BlockSpec((blk, 1), lambda i: (i, 0)),
            pl.BlockSpec((C, C), lambda i: (0, 0)),
        ],
        out_specs=pl.BlockSpec((blk, C), lambda i: (i, 0)),
        out_shape=jax.ShapeDtypeStruct((N_N, C), jnp.float32),
    )


@functools.cache
def _tc_hedge():
    blk = 2000

    def body(pa_ref, sp_ref, dea_ref, b_ref, w_ref, o_ref):
        seg = pa_ref[0] + pa_ref[1]
        s1 = jnp.sum(sp_ref[...], axis=1, keepdims=True)
        inv = jnp.where(s1 > 0, 1.0 / s1, 0.0)
        x1 = jnp.maximum(seg * inv + b_ref[...], 0.0)
        o_ref[...] = jnp.dot(x1 * dea_ref[...], w_ref[...],
                             preferred_element_type=jnp.float32)

    return pl.pallas_call(
        body,
        grid=(N_H // blk,),
        in_specs=[
            pl.BlockSpec((2, blk, C), lambda i: (0, i, 0)),
            pl.BlockSpec((blk, 16), lambda i: (i, 0)),
            pl.BlockSpec((blk, 1), lambda i: (i, 0)),
            pl.BlockSpec((1, C), lambda i: (0, 0)),
            pl.BlockSpec((C, C), lambda i: (0, 0)),
        ],
        out_specs=pl.BlockSpec((blk, C), lambda i: (i, 0)),
        out_shape=jax.ShapeDtypeStruct((N_H, C), jnp.float32),
    )


@functools.cache
def _tc_node():
    blk = 2000

    def body(seg_ref, sp_ref, b_ref, sc_ref, w_ref, o_ref):
        s0 = jnp.sum(sp_ref[...], axis=1, keepdims=True)
        inv = jnp.where(s0 > 0, 1.0 / s0, 0.0)
        x = jnp.maximum(seg_ref[...] * inv + b_ref[...], 0.0)
        o_ref[...] = jnp.dot(x * sc_ref[...], w_ref[...],
                             preferred_element_type=jnp.float32)

    return pl.pallas_call(
        body,
        grid=(N_N // blk,),
        in_specs=[
            pl.BlockSpec((blk, C), lambda i: (i, 0)),
            pl.BlockSpec((blk, 16), lambda i: (i, 0)),
            pl.BlockSpec((1, C), lambda i: (0, 0)),
            pl.BlockSpec((blk, 1), lambda i: (i, 0)),
            pl.BlockSpec((C, C), lambda i: (0, 0)),
        ],
        out_specs=pl.BlockSpec((blk, C), lambda i: (i, 0)),
        out_shape=jax.ShapeDtypeStruct((N_N, C), jnp.float32),
    )


@functools.cache
def _tc_final():
    blk = 2000
    ngrid = N_N // blk

    def body(seg_ref, sp_ref, b_ref, wl_ref, bl_ref, o_ref, pool_ref):
        s0 = jnp.sum(sp_ref[...], axis=1, keepdims=True)
        inv = jnp.where(s0 > 0, 1.0 / s0, 0.0)
        x = jnp.maximum(seg_ref[...] * inv + b_ref[...], 0.0)
        bm = jnp.max(x, axis=0, keepdims=True)
        i = pl.program_id(0)

        @pl.when(i == 0)
        def _():
            pool_ref[...] = bm

        @pl.when(i > 0)
        def _():
            pool_ref[...] = jnp.maximum(pool_ref[...], bm)

        @pl.when(i == ngrid - 1)
        def _():
            o_ref[...] = jnp.dot(pool_ref[...], wl_ref[...],
                                 preferred_element_type=jnp.float32) + bl_ref[...]

    return pl.pallas_call(
        body,
        grid=(ngrid,),
        in_specs=[
            pl.BlockSpec((blk, C), lambda i: (i, 0)),
            pl.BlockSpec((blk, 16), lambda i: (i, 0)),
            pl.BlockSpec((1, C), lambda i: (0, 0)),
            pl.BlockSpec((C, 1), lambda i: (0, 0)),
            pl.BlockSpec((1, 1), lambda i: (0, 0)),
        ],
        out_specs=pl.BlockSpec((1, 1), lambda i: (0, 0)),
        out_shape=jax.ShapeDtypeStruct((1, 1), jnp.float32),
        scratch_shapes=[pltpu.VMEM((1, C), jnp.float32)],
    )


def _pad_to(a, n, val):
    return jnp.concatenate(
        [a, jnp.full((n - a.shape[0],), val, a.dtype)])


def kernel(x_0, node_idx, hedge_idx,
           W01_1, b01_1, W10_1, b10_1,
           W01_2, b01_2, W10_2, b10_2,
           W_lin, b_lin):
    ni = node_idx.astype(jnp.int32)
    hi = hedge_idx.astype(jnp.int32)

    src_a = _pad_to(ni, RPAD, 0)
    dst_a = _pad_to(hi, RPAD, N_H).reshape(RROWS, 128)
    src_b = _pad_to(hi, RPAD, 0)
    nip = _pad_to(ni, RPAD, N_N)
    dst_blo = jnp.where(nip < DBH, nip, DBH)
    nih = nip - DBH
    dst_bhi = jnp.where((nih >= 0) & (nih < DBH), nih, DBH)
    hc = _pad_to(hi, CNT_PAD, N_H)
    nc = _pad_to(ni, CNT_PAD, N_N)

    de_p, dv_p = _sc_counts()(hc, nc)
    dea_t, dvb_t = _tc_scales()(de_p.reshape(32, HN), dv_p.reshape(32, NN))
    s0_p, s1_p = _sc_ssums()(hc, nc, dea_t[0], dvb_t[0])
    s0_p = s0_p.reshape(16, NN)
    s1_p = s1_p.reshape(16, HN)

    s0t = s0_p[:, :N_N].T
    s1t = s1_p[:, :N_H].T
    dea_col = dea_t[0, :N_H].reshape(N_H, 1)
    dvb_col = dvb_t[0, :N_N].reshape(N_N, 1)
    b01_1r = b01_1.reshape(1, C)
    b10_1r = b10_1.reshape(1, C)
    b01_2r = b01_2.reshape(1, C)
    b10_2r = b10_2.reshape(1, C)

    m = _tc_pre()(x_0, dvb_col, W01_1)
    pa = _sc_seg_hedge()(m, src_a, dst_a).reshape(2, HN, C)[:, :N_H]
    m1 = _tc_hedge()(pa, s1t, dea_col, b01_1r, W10_1)
    segb = _sc_seg_node()(m1, src_b, dst_blo, dst_bhi)
    m2 = _tc_node()(segb, s0t, b10_1r, dvb_col, W01_2)
    pa2 = _sc_seg_hedge()(m2, src_a, dst_a).reshape(2, HN, C)[:, :N_H]
    m3 = _tc_hedge()(pa2, s1t, dea_col, b01_2r, W10_2)
    segb2 = _sc_seg_node()(m3, src_b, dst_blo, dst_bhi)
    out = _tc_final()(segb2, s0t, b10_2r, W_lin, b_lin.reshape(1, 1))
    return out.reshape(1)


# speedup vs baseline: 1.5047x; 1.0294x over previous
import functools

import jax
import jax.numpy as jnp
from jax import lax
from jax.experimental import pallas as pl
from jax.experimental.pallas import tpu as pltpu
from jax.experimental.pallas import tpu_sc as plsc

N_N = 50000
N_H = 10000
NNZ = 800000
C = 64

CH = 512
EPT_A = 25600
ITER_A = EPT_A // 1024
RPAD = 32 * EPT_A
RROWS = RPAD // 128
EPT_B = 2 * EPT_A
ITER_B = 2 * ITER_A

CNT_CH = 1024
CNT_PAD = 819200
CNT_EPT = CNT_PAD // 32
CNT_CPT = CNT_EPT // CNT_CH
SS_EPT = CNT_PAD // 16
SS_CPT = SS_EPT // CNT_CH

HN = 10112
NN = 50016
DBH = N_N // 2
DBA = 25088


def _mesh():
    return plsc.VectorSubcoreMesh(core_axis_name="c", subcore_axis_name="s")


def _zero16():
    return jnp.zeros((16,), jnp.float32)


@functools.cache
def _sc_counts():
    @functools.partial(
        pl.kernel,
        out_type=(
            jax.ShapeDtypeStruct((32 * HN,), jnp.float32),
            jax.ShapeDtypeStruct((32 * NN,), jnp.float32),
        ),
        mesh=_mesh(),
        compiler_params=pltpu.CompilerParams(needs_layout_passes=False, use_tc_tiling_on_sc=False),
        scratch_types=[
            pltpu.VMEM((HN,), jnp.float32),
            pltpu.VMEM((NN,), jnp.float32),
            pltpu.VMEM((CNT_EPT,), jnp.int32),
            pltpu.VMEM((CNT_EPT,), jnp.int32),
            pltpu.SemaphoreType.DMA,
        ],
    )
    def k(hc_hbm, nc_hbm, de_p, dv_p, acc_de, acc_dv, hbuf, nbuf, sem):
        c = lax.axis_index("c")
        s = lax.axis_index("s")
        w = c * 16 + s
        z = _zero16()
        ones = jnp.ones((16,), jnp.float32)

        d1 = pltpu.async_copy(hc_hbm.at[pl.ds(w * CNT_EPT, CNT_EPT)], hbuf,
                              sem)
        d2 = pltpu.async_copy(nc_hbm.at[pl.ds(w * CNT_EPT, CNT_EPT)], nbuf,
                              sem)

        @pl.loop(0, HN // 16)
        def _(i):
            acc_de[pl.ds(i * 16, 16)] = z

        @pl.loop(0, NN // 16)
        def _(i):
            acc_dv[pl.ds(i * 16, 16)] = z

        d1.wait()
        d2.wait()

        @pl.loop(0, CNT_EPT // 16)
        def _(i):
            hv = hbuf[pl.ds(i * 16, 16)]
            plsc.addupdate_scatter(acc_de, [hv], ones)
            nv = nbuf[pl.ds(i * 16, 16)]
            plsc.addupdate_scatter(acc_dv, [nv], ones)

        pltpu.sync_copy(acc_de, de_p.at[pl.ds(w * HN, HN)])
        pltpu.sync_copy(acc_dv, dv_p.at[pl.ds(w * NN, NN)])

    return k


@functools.cache
def _sc_ssums():
    @functools.partial(
        pl.kernel,
        out_type=(
            jax.ShapeDtypeStruct((16 * NN,), jnp.float32),
            jax.ShapeDtypeStruct((16 * HN,), jnp.float32),
        ),
        mesh=_mesh(),
        compiler_params=pltpu.CompilerParams(needs_layout_passes=False, use_tc_tiling_on_sc=False),
        scratch_types=[
            pltpu.VMEM((NN,), jnp.float32),
            pltpu.VMEM((NN,), jnp.float32),
            pltpu.VMEM((SS_EPT // 4,), jnp.int32),
            pltpu.VMEM((SS_EPT // 4,), jnp.int32),
            pltpu.SemaphoreType.DMA,
        ],
    )
    def k(hc_hbm, nc_hbm, dea_hbm, dvb_hbm, s0_p, s1_p,
          tbl, acc, hbuf, nbuf, sem):
        c = lax.axis_index("c")
        s = lax.axis_index("s")
        z = _zero16()
        qn = SS_EPT // 4

        @pl.when(c == 0)
        def _():
            pltpu.sync_copy(dea_hbm, tbl.at[pl.ds(0, HN)])

        @pl.when(c == 1)
        def _():
            pltpu.sync_copy(dvb_hbm, tbl)

        @pl.loop(0, NN // 16)
        def _(i):
            acc[pl.ds(i * 16, 16)] = z

        @pl.loop(0, 4)
        def _(q):
            b = s * SS_EPT + q * qn
            d1 = pltpu.async_copy(hc_hbm.at[pl.ds(b, qn)], hbuf, sem)
            d2 = pltpu.async_copy(nc_hbm.at[pl.ds(b, qn)], nbuf, sem)
            d1.wait()
            d2.wait()

            @pl.when(c == 0)
            def _():
                @pl.loop(0, qn // 16)
                def _(i):
                    hv = hbuf[pl.ds(i * 16, 16)]
                    nv = nbuf[pl.ds(i * 16, 16)]
                    val = plsc.load_gather(tbl, [hv])
                    plsc.addupdate_scatter(acc, [nv], val)

            @pl.when(c == 1)
            def _():
                @pl.loop(0, qn // 16)
                def _(i):
                    hv = hbuf[pl.ds(i * 16, 16)]
                    nv = nbuf[pl.ds(i * 16, 16)]
                    val = plsc.load_gather(tbl, [nv])
                    plsc.addupdate_scatter(acc, [hv], val)

        @pl.when(c == 0)
        def _():
            pltpu.sync_copy(acc, s0_p.at[pl.ds(s * NN, NN)])

        @pl.when(c == 1)
        def _():
            pltpu.sync_copy(acc.at[pl.ds(0, HN)], s1_p.at[pl.ds(s * HN, HN)])

    return k


def _zero_rows(rows_v, n):
    z = _zero16()

    @pl.loop(0, n)
    def _(i):
        for q in range(4):
            rows_v[i, pl.ds(q * 16, 16)] = z


@functools.cache
def _sc_seg_hedge():
    @functools.partial(
        pl.kernel,
        out_type=jax.ShapeDtypeStruct((2 * HN, C), jnp.float32),
        mesh=_mesh(),
        compiler_params=pltpu.CompilerParams(needs_layout_passes=False, use_tc_tiling_on_sc=False),
        scratch_types=[
            pltpu.VMEM((5120,), jnp.int32),
            pltpu.VMEM((40, 128), jnp.int32),
            pltpu.VMEM((1024, C), jnp.float32),
            pltpu.VMEM_SHARED((HN, C), jnp.float32),
            pltpu.SemaphoreType.DMA,
            pltpu.SemaphoreType.DMA,
        ],
    )
    def k(in_hbm, src_hbm, dst_hbm, out_hbm,
          src_v, idx_v, rows_v, acc, semg, sems):
        c = lax.axis_index("c")
        s = lax.axis_index("s")
        w = c * 16 + s

        _zero_rows(rows_v, 256)
        st = s * (HN // 16)
        for t in range(2):
            pltpu.sync_copy(rows_v.at[pl.ds(0, 256)],
                            acc.at[pl.ds(st + t * 256, 256)])
        pltpu.sync_copy(rows_v.at[pl.ds(0, 120)], acc.at[pl.ds(st + 512, 120)])
        plsc.subcore_barrier()

        @pl.loop(0, ITER_A // 5)
        def _(u):
            eb = w * EPT_A + u * 5120
            rb = w * (EPT_A // 128) + u * 40
            d1 = pltpu.async_copy(src_hbm.at[pl.ds(eb, 5120)], src_v, semg)
            d2 = pltpu.async_copy(dst_hbm.at[pl.ds(rb, 40)], idx_v, semg)
            d1.wait()
            d2.wait()
            _pipe_super(in_hbm, acc, src_v, idx_v, rows_v, semg, sems, 512)

        plsc.subcore_barrier()
        pltpu.sync_copy(acc.at[pl.ds(st, CH)],
                        out_hbm.at[pl.ds(c * HN + st, CH)])
        pltpu.sync_copy(acc.at[pl.ds(st + CH, HN // 16 - CH)],
                        out_hbm.at[pl.ds(c * HN + st + CH, HN // 16 - CH)])

    return k


def _pipe_super(in_hbm, acc, src_v, idx_v, rows_v, semg, sems, psz):
    npart = 5120 // psz
    nsc = psz // 128

    def issue_gather(p, off):
        pltpu.async_copy(in_hbm.at[src_v.at[pl.ds(p * psz, psz)]],
                         rows_v.at[pl.ds(off, psz)], semg)

    def wait_gather(p, off):
        pltpu.make_async_copy(in_hbm.at[src_v.at[pl.ds(p * psz, psz)]],
                              rows_v.at[pl.ds(off, psz)], semg).wait()

    def issue_scatters(p, off):
        for j in range(nsc):
            pltpu.async_copy(rows_v.at[pl.ds(off + j * 128, 128)],
                             acc.at[idx_v.at[p * nsc + j]], sems, add=True)

    def wait_scatters(p, off):
        for j in range(nsc):
            pltpu.make_async_copy(rows_v.at[pl.ds(off + j * 128, 128)],
                                  acc.at[idx_v.at[p * nsc + j]],
                                  sems).wait()

    issue_gather(0, 0)

    @pl.loop(0, npart)
    def _(p):
        b = (p % 2) * psz
        nb = psz - b

        @pl.when(p >= 1)
        def _():
            wait_scatters(p - 1, nb)

        @pl.when(p < npart - 1)
        def _():
            issue_gather(p + 1, nb)

        wait_gather(p, b)
        issue_scatters(p, b)

    wait_scatters(npart - 1, ((npart - 1) % 2) * psz)


@functools.cache
def _sc_seg_node():
    @functools.partial(
        pl.kernel,
        out_type=jax.ShapeDtypeStruct((N_N, C), jnp.float32),
        mesh=_mesh(),
        compiler_params=pltpu.CompilerParams(needs_layout_passes=False, use_tc_tiling_on_sc=False),
        scratch_types=[
            pltpu.VMEM((5248,), jnp.int32),
            pltpu.VMEM((5248,), jnp.int32),
            pltpu.VMEM((256, C), jnp.float32),
            pltpu.VMEM_SHARED((DBA, C), jnp.float32),
            pltpu.SemaphoreType.DMA,
            pltpu.SemaphoreType.DMA,
        ],
    )
    def k(in_hbm, src_hbm, dlo_hbm, dhi_hbm, out_hbm,
          src_v, dst_v, rows_v, acc, semg, sems):
        c = lax.axis_index("c")
        s = lax.axis_index("s")
        lo = c * DBH

        _zero_rows(rows_v, 256)
        st = s * (DBA // 16)
        for t in range(6):
            pltpu.sync_copy(rows_v, acc.at[pl.ds(st + t * 256, 256)])
        pltpu.sync_copy(rows_v.at[pl.ds(0, 32)], acc.at[pl.ds(st + 1536, 32)])
        plsc.subcore_barrier()

        def gat(p, off):
            return pltpu.async_copy(
                in_hbm.at[src_v.at[pl.ds(p * 128, 128)]],
                rows_v.at[pl.ds(off, 128)], semg)

        def wgat(p, off):
            pltpu.make_async_copy(
                in_hbm.at[src_v.at[pl.ds(p * 128, 128)]],
                rows_v.at[pl.ds(off, 128)], semg).wait()

        def sca(p, off):
            return pltpu.async_copy(
                rows_v.at[pl.ds(off, 128)],
                acc.at[dst_v.at[pl.ds(p * 128, 128)]], sems, add=True)

        def wsca(p, off):
            pltpu.make_async_copy(
                rows_v.at[pl.ds(off, 128)],
                acc.at[dst_v.at[pl.ds(p * 128, 128)]], sems).wait()

        def main(dst_hbm):
            @pl.loop(0, ITER_B // 5)
            def _(u):
                eb = s * EPT_B + u * 5120
                d1 = pltpu.async_copy(src_hbm.at[pl.ds(eb, 5120)],
                                      src_v.at[pl.ds(0, 5120)], semg)
                d2 = pltpu.async_copy(dst_hbm.at[pl.ds(eb, 5120)],
                                      dst_v.at[pl.ds(0, 5120)], semg)
                d1.wait()
                d2.wait()

                @pl.loop(0, 40, init_carry=0)
                def n(r, cur):
                    for t in range(8):
                        off = r * 128 + t * 16
                        dv = dst_v[pl.ds(off, 16)]
                        sv = src_v[pl.ds(off, 16)]
                        msk = dv != DBH
                        plsc.store_compressed(dst_v.at[pl.ds(cur, 16)], dv,
                                              mask=msk)
                        plsc.store_compressed(src_v.at[pl.ds(cur, 16)], sv,
                                              mask=msk)
                        cur = cur + plsc.all_reduce_population_count(msk)[0]
                    return cur

                for t in range(8):
                    dst_v[pl.ds(n + t * 16, 16)] = jnp.full((16,), DBH,
                                                            jnp.int32)
                    src_v[pl.ds(n + t * 16, 16)] = jnp.zeros((16,), jnp.int32)
                nparts = (n + 127) // 128

                @pl.when(nparts > 0)
                def _():
                    gat(0, 0)

                    @pl.loop(0, nparts)
                    def _(p):
                        b = (p % 2) * 128
                        nb = 128 - b

                        @pl.when(p >= 1)
                        def _():
                            wsca(p - 1, nb)

                        @pl.when(p < nparts - 1)
                        def _():
                            gat(p + 1, nb)

                        wgat(p, b)
                        sca(p, b)

                    wsca(nparts - 1, ((nparts - 1) % 2) * 128)

        @pl.when(c == 0)
        def _():
            main(dlo_hbm)

        @pl.when(c == 1)
        def _():
            main(dhi_hbm)

        plsc.subcore_barrier()
        st15 = s * 1560
        for t in range(3):
            pltpu.sync_copy(acc.at[pl.ds(st15 + t * CH, CH)],
                            out_hbm.at[pl.ds(lo + st15 + t * CH, CH)])
        pltpu.sync_copy(acc.at[pl.ds(st15 + 3 * CH, 24)],
                        out_hbm.at[pl.ds(lo + st15 + 3 * CH, 24)])

        @pl.when(s == 0)
        def _():
            pltpu.sync_copy(acc.at[pl.ds(24960, 40)],
                            out_hbm.at[pl.ds(lo + 24960, 40)])

    return k


@functools.cache
def _tc_scales():
    def body(de_ref, dv_ref, dea_ref, dvb_ref):
        de = jnp.sum(de_ref[...], axis=0, keepdims=True)
        r = lax.rsqrt(de)
        dea_ref[...] = jnp.where(de > 0, r * r * r, 0.0)
        dv = jnp.sum(dv_ref[...], axis=0, keepdims=True)
        r2 = lax.rsqrt(dv)
        dvb_ref[...] = jnp.where(dv > 0, r2, 0.0)

    return pl.pallas_call(
        body,
        out_shape=(
            jax.ShapeDtypeStruct((1, HN), jnp.float32),
            jax.ShapeDtypeStruct((1, NN), jnp.float32),
        ),
    )


@functools.cache
def _tc_pre():
    blk = 2000

    def body(x_ref, sc_ref, w_ref, o_ref):
        o_ref[...] = jnp.dot(x_ref[...] * sc_ref[...], w_ref[...],
                             preferred_element_type=jnp.float32)

    return pl.pallas_call(
        body,
        grid=(N_N // blk,),
        in_specs=[
            pl.BlockSpec((blk, C), lambda i: (i, 0)),
            pl.BlockSpec((blk, 1), lambda i: (i, 0)),
            pl.BlockSpec((C, C), lambda i: (0, 0)),
        ],
        out_specs=pl.BlockSpec((blk, C), lambda i: (i, 0)),
        out_shape=jax.ShapeDtypeStruct((N_N, C), jnp.float32),
    )


@functools.cache
def _tc_hedge():
    blk = 2000

    def body(pa_ref, sp_ref, dea_ref, b_ref, w_ref, o_ref):
        seg = pa_ref[0] + pa_ref[1]
        s1 = jnp.sum(sp_ref[...], axis=1, keepdims=True)
        inv = jnp.where(s1 > 0, 1.0 / s1, 0.0)
        x1 = jnp.maximum(seg * inv + b_ref[...], 0.0)
        o_ref[...] = jnp.dot(x1 * dea_ref[...], w_ref[...],
                             preferred_element_type=jnp.float32)

    return pl.pallas_call(
        body,
        grid=(N_H // blk,),
        in_specs=[
            pl.BlockSpec((2, blk, C), lambda i: (0, i, 0)),
            pl.BlockSpec((blk, 16), lambda i: (i, 0)),
            pl.BlockSpec((blk, 1), lambda i: (i, 0)),
            pl.BlockSpec((1, C), lambda i: (0, 0)),
            pl.BlockSpec((C, C), lambda i: (0, 0)),
        ],
        out_specs=pl.BlockSpec((blk, C), lambda i: (i, 0)),
        out_shape=jax.ShapeDtypeStruct((N_H, C), jnp.float32),
    )


@functools.cache
def _tc_node():
    blk = 2000

    def body(seg_ref, sp_ref, b_ref, sc_ref, w_ref, o_ref):
        s0 = jnp.sum(sp_ref[...], axis=1, keepdims=True)
        inv = jnp.where(s0 > 0, 1.0 / s0, 0.0)
        x = jnp.maximum(seg_ref[...] * inv + b_ref[...], 0.0)
        o_ref[...] = jnp.dot(x * sc_ref[...], w_ref[...],
                             preferred_element_type=jnp.float32)

    return pl.pallas_call(
        body,
        grid=(N_N // blk,),
        in_specs=[
            pl.BlockSpec((blk, C), lambda i: (i, 0)),
            pl.BlockSpec((blk, 16), lambda i: (i, 0)),
            pl.BlockSpec((1, C), lambda i: (0, 0)),
            pl.BlockSpec((blk, 1), lambda i: (i, 0)),
            pl.BlockSpec((C, C), lambda i: (0, 0)),
        ],
        out_specs=pl.BlockSpec((blk, C), lambda i: (i, 0)),
        out_shape=jax.ShapeDtypeStruct((N_N, C), jnp.float32),
    )


@functools.cache
def _tc_final():
    blk = 2000
    ngrid = N_N // blk

    def body(seg_ref, sp_ref, b_ref, wl_ref, bl_ref, o_ref, pool_ref):
        s0 = jnp.sum(sp_ref[...], axis=1, keepdims=True)
        inv = jnp.where(s0 > 0, 1.0 / s0, 0.0)
        x = jnp.maximum(seg_ref[...] * inv + b_ref[...], 0.0)
        bm = jnp.max(x, axis=0, keepdims=True)
        i = pl.program_id(0)

        @pl.when(i == 0)
        def _():
            pool_ref[...] = bm

        @pl.when(i > 0)
        def _():
            pool_ref[...] = jnp.maximum(pool_ref[...], bm)

        @pl.when(i == ngrid - 1)
        def _():
            o_ref[...] = jnp.dot(pool_ref[...], wl_ref[...],
                                 preferred_element_type=jnp.float32) + bl_ref[...]

    return pl.pallas_call(
        body,
        grid=(ngrid,),
        in_specs=[
            pl.BlockSpec((blk, C), lambda i: (i, 0)),
            pl.BlockSpec((blk, 16), lambda i: (i, 0)),
            pl.BlockSpec((1, C), lambda i: (0, 0)),
            pl.BlockSpec((C, 1), lambda i: (0, 0)),
            pl.BlockSpec((1, 1), lambda i: (0, 0)),
        ],
        out_specs=pl.BlockSpec((1, 1), lambda i: (0, 0)),
        out_shape=jax.ShapeDtypeStruct((1, 1), jnp.float32),
        scratch_shapes=[pltpu.VMEM((1, C), jnp.float32)],
    )


def _pad_to(a, n, val):
    return jnp.concatenate(
        [a, jnp.full((n - a.shape[0],), val, a.dtype)])


def kernel(x_0, node_idx, hedge_idx,
           W01_1, b01_1, W10_1, b10_1,
           W01_2, b01_2, W10_2, b10_2,
           W_lin, b_lin):
    ni = node_idx.astype(jnp.int32)
    hi = hedge_idx.astype(jnp.int32)

    src_a = _pad_to(ni, RPAD, 0)
    dst_a = _pad_to(hi, RPAD, N_H).reshape(RROWS, 128)
    src_b = _pad_to(hi, RPAD, 0)
    nip = _pad_to(ni, RPAD, N_N)
    dst_blo = jnp.where(nip < DBH, nip, DBH)
    nih = nip - DBH
    dst_bhi = jnp.where((nih >= 0) & (nih < DBH), nih, DBH)
    hc = _pad_to(hi, CNT_PAD, N_H)
    nc = _pad_to(ni, CNT_PAD, N_N)

    de_p, dv_p = _sc_counts()(hc, nc)
    dea_t, dvb_t = _tc_scales()(de_p.reshape(32, HN), dv_p.reshape(32, NN))
    s0_p, s1_p = _sc_ssums()(hc, nc, dea_t[0], dvb_t[0])
    s0_p = s0_p.reshape(16, NN)
    s1_p = s1_p.reshape(16, HN)

    s0t = s0_p[:, :N_N].T
    s1t = s1_p[:, :N_H].T
    dea_col = dea_t[0, :N_H].reshape(N_H, 1)
    dvb_col = dvb_t[0, :N_N].reshape(N_N, 1)
    b01_1r = b01_1.reshape(1, C)
    b10_1r = b10_1.reshape(1, C)
    b01_2r = b01_2.reshape(1, C)
    b10_2r = b10_2.reshape(1, C)

    m = _tc_pre()(x_0, dvb_col, W01_1)
    pa = _sc_seg_hedge()(m, src_a, dst_a).reshape(2, HN, C)[:, :N_H]
    m1 = _tc_hedge()(pa, s1t, dea_col, b01_1r, W10_1)
    segb = _sc_seg_node()(m1, src_b, dst_blo, dst_bhi)
    m2 = _tc_node()(segb, s0t, b10_1r, dvb_col, W01_2)
    pa2 = _sc_seg_hedge()(m2, src_a, dst_a).reshape(2, HN, C)[:, :N_H]
    m3 = _tc_hedge()(pa2, s1t, dea_col, b01_2r, W10_2)
    segb2 = _sc_seg_node()(m3, src_b, dst_blo, dst_bhi)
    out = _tc_final()(segb2, s0t, b10_2r, W_lin, b_lin.reshape(1, 1))
    return out.reshape(1)
